# baseline (speedup 1.0000x reference)
import functools

import jax
import jax.numpy as jnp
import numpy as _np
from jax import lax
from jax.experimental import pallas as pl
from jax.experimental.pallas import tpu as pltpu
from jax.experimental.pallas import tpu_sc as plsc

_VOCAB = 1000000
_D = 64
_ODIM = 128
_B = 16384
_CTX = 50

_NC = 2
_NS = 16
_NW = _NC * _NS
_BPW = _B // _NW
_CB = 8
_NCHUNK = _BPW // _CB

_VBLK = 512
_NBLK = -(-_VOCAB // _VBLK)
_VPAD = _NBLK * _VBLK

_PERM = _np.concatenate([
    _np.arange(0, 32, 2), _np.arange(1, 32, 2),
    _np.arange(32, 64, 2), _np.arange(33, 64, 2),
])


def _tc_relayout(embedT):
    def body(x_ref, out_ref, fe, fo, e0, e1, e2, e3):
        i = pl.program_id(0)
        es = (e0, e1, e2, e3)

        @pl.when(i == 0)
        def _():
            jf = lax.broadcasted_iota(jnp.int32, (32, _D), 0)
            ff = lax.broadcasted_iota(jnp.int32, (32, _D), 1)
            fe[...] = (ff == 2 * jf).astype(jnp.bfloat16)
            fo[...] = (ff == 2 * jf + 1).astype(jnp.bfloat16)
            kk = lax.broadcasted_iota(jnp.int32, (_VBLK, 128), 0)
            rr = lax.broadcasted_iota(jnp.int32, (_VBLK, 128), 1)
            for m in range(4):
                es[m][...] = (kk == 4 * rr + m).astype(jnp.bfloat16)

        x16 = x_ref[...].astype(jnp.bfloat16)
        dn_f = (((1,), (0,)), ((), ()))
        ue = lax.dot_general(fe[...], x16, dn_f,
                             preferred_element_type=jnp.float32)
        uo = lax.dot_general(fo[...], x16, dn_f,
                             preferred_element_type=jnp.float32)
        ue16 = ue.astype(jnp.bfloat16)
        uo16 = uo.astype(jnp.bfloat16)
        dn_v = (((0,), (1,)), ((), ()))
        parts = []
        for m in range(4):
            lo = lax.dot_general(es[m][...], ue16, dn_v,
                                 preferred_element_type=jnp.float32)
            hi = lax.dot_general(es[m][...], uo16, dn_v,
                                 preferred_element_type=jnp.float32)
            lo_w = lax.bitcast_convert_type(
                lo.astype(jnp.bfloat16), jnp.int16).astype(jnp.int32) & 0xFFFF
            hi_w = lax.shift_left(
                lax.bitcast_convert_type(
                    hi.astype(jnp.bfloat16), jnp.int16).astype(jnp.int32), 16)
            parts.append(lo_w | hi_w)
        out_ref[...] = jnp.concatenate(parts, axis=1)

    return pl.pallas_call(
        body,
        grid=(_NBLK,),
        in_specs=[pl.BlockSpec((_D, _VBLK), lambda i: (0, i))],
        out_specs=pl.BlockSpec((128, 128), lambda i: (i, 0)),
        out_shape=jax.ShapeDtypeStruct((_VPAD // 4, 128), jnp.int32),
        scratch_shapes=[
            pltpu.VMEM((32, _D), jnp.bfloat16),
            pltpu.VMEM((32, _D), jnp.bfloat16),
        ] + [pltpu.VMEM((_VBLK, 128), jnp.bfloat16) for _ in range(4)],
    )(embedT)


def _sc_pool(idx2d, table_w):
    mesh = plsc.VectorSubcoreMesh(core_axis_name="c", subcore_axis_name="s")

    @functools.partial(
        pl.kernel,
        mesh=mesh,
        compiler_params=pltpu.CompilerParams(
            use_tc_tiling_on_sc=False, needs_layout_passes=False),
        out_type=jax.ShapeDtypeStruct((_B, _D), jnp.float32),
        scratch_types=[
            pltpu.VMEM((_BPW, _CTX), jnp.int32),
            pltpu.VMEM((2, _CB, _CTX, _D // 2), jnp.int32),
            pltpu.VMEM((2, _CB, _D), jnp.float32),
            pltpu.SemaphoreType.DMA,
            pltpu.SemaphoreType.DMA,
            pltpu.SemaphoreType.DMA,
            pltpu.SemaphoreType.DMA,
        ],
    )
    def k(idx_hbm, table_hbm, out_hbm, idx_v, rows_v, acc_v, g0, g1, o0, o1):
        wid = lax.axis_index("s") * _NC + lax.axis_index("c")
        b0w = wid * _BPW
        gsem = (g0, g1)
        osem = (o0, o1)

        pltpu.sync_copy(idx_hbm.at[pl.ds(b0w, _BPW)], idx_v)

        def gather_descs(i, par):
            return [
                pltpu.make_async_copy(
                    table_hbm.at[idx_v.at[i * _CB + bb]],
                    rows_v.at[par, bb],
                    gsem[par],
                )
                for bb in range(_CB)
            ]

        def fire(i, par):
            for d in gather_descs(i, par):
                d.start()

        def drain(i, par):
            for d in gather_descs(i, par):
                d.wait()

        def pool(i, par):
            for bb in range(_CB):
                zeros = jnp.zeros((16,), jnp.float32)

                def ctx_body(c, acc, bb=bb, par=par):
                    r = c * 2
                    a0, a1, a2, a3 = acc
                    for u in range(2):
                        lohalf = plsc.bitcast(
                            rows_v[par, bb, r + u, pl.ds(0, 16)], jnp.bfloat16)
                        hihalf = plsc.bitcast(
                            rows_v[par, bb, r + u, pl.ds(16, 16)], jnp.bfloat16)
                        e0, o0_ = plsc.unpack(
                            lohalf, format=plsc.PackFormat.INTERLEAVED)
                        e1, o1_ = plsc.unpack(
                            hihalf, format=plsc.PackFormat.INTERLEAVED)
                        a0 = a0 + e0
                        a1 = a1 + o0_
                        a2 = a2 + e1
                        a3 = a3 + o1_
                    return (a0, a1, a2, a3)

                a0, a1, a2, a3 = lax.fori_loop(
                    0, _CTX // 2, ctx_body, (zeros, zeros, zeros, zeros))
                acc_v[par, bb, pl.ds(0, 16)] = a0
                acc_v[par, bb, pl.ds(16, 16)] = a1
                acc_v[par, bb, pl.ds(32, 16)] = a2
                acc_v[par, bb, pl.ds(48, 16)] = a3

        def out_desc(i, par):
            return pltpu.make_async_copy(
                acc_v.at[par],
                out_hbm.at[pl.ds(b0w + i * _CB, _CB)],
                osem[par],
            )

        fire(0, 0)

        def pair_body(p, carry):
            for q in range(2):
                i = 2 * p + q
                par = q
                drain(i, par)

                @pl.when(i + 1 < _NCHUNK)
                def _():
                    fire(i + 1, 1 - par)

                @pl.when(i >= 2)
                def _():
                    out_desc(i - 2, par).wait()

                pool(i, par)
                out_desc(i, par).start()
            return carry

        lax.fori_loop(0, _NCHUNK // 2, pair_body, 0)

        out_desc(_NCHUNK - 2, 0).wait()
        out_desc(_NCHUNK - 1, 1).wait()

    return k(idx2d, table_w)


def _tc_linear(pooled, Wp, b2d):
    BB = 2048

    def body(x_ref, w_ref, b_ref, o_ref):
        o_ref[...] = lax.dot_general(
            x_ref[...], w_ref[...], (((1,), (1,)), ((), ())),
            preferred_element_type=jnp.float32,
        ) + b_ref[...]

    return pl.pallas_call(
        body,
        grid=(_B // BB,),
        in_specs=[
            pl.BlockSpec((BB, _D), lambda i: (i, 0)),
            pl.BlockSpec((_ODIM, _D), lambda i: (0, 0)),
            pl.BlockSpec((1, _ODIM), lambda i: (0, 0)),
        ],
        out_specs=pl.BlockSpec((BB, _ODIM), lambda i: (i, 0)),
        out_shape=jax.ShapeDtypeStruct((_B, _ODIM), jnp.float32),
    )(pooled, Wp, b2d)


def kernel(inputs, embed, W, b):
    packed = _tc_relayout(jnp.transpose(embed))
    pooled = _sc_pool(inputs.astype(jnp.int32), packed.reshape(_VPAD, _D // 2))
    Wp = W[:, _PERM]
    return _tc_linear(pooled, Wp, b.reshape(1, _ODIM))


# speedup vs baseline: 3.5543x; 3.5543x over previous
import functools

import jax
import jax.numpy as jnp
import numpy as _np
from jax import lax
from jax.experimental import pallas as pl
from jax.experimental.pallas import tpu as pltpu
from jax.experimental.pallas import tpu_sc as plsc

_VOCAB = 1000000
_D = 64
_ODIM = 128
_B = 16384
_CTX = 50

_NC = 2
_NS = 16
_NW = _NC * _NS
_BPW = _B // _NW
_CB = 8
_NCHUNK = _BPW // _CB

_QW = 1024
_NBLK = 245
_VQ = _NBLK * _QW
_VPAD = 4 * _VQ

_PERM = _np.concatenate([
    _np.arange(0, 32, 2), _np.arange(1, 32, 2),
    _np.arange(32, 64, 2), _np.arange(33, 64, 2),
])


def _tc_relayout(embedT):
    def body(x0, x1, x2, x3, out_ref, i_scr, fe, fo):
        i = pl.program_id(0)

        @pl.when(i == 0)
        def _():
            a = lax.broadcasted_iota(jnp.int32, (_D, _D), 0)
            b = lax.broadcasted_iota(jnp.int32, (_D, _D), 1)
            i_scr[...] = (a == b).astype(jnp.bfloat16)
            jf = lax.broadcasted_iota(jnp.int32, (_D, 32), 0)
            ff = lax.broadcasted_iota(jnp.int32, (_D, 32), 1)
            fe[...] = (jf == 2 * ff).astype(jnp.bfloat16)
            fo[...] = (jf == 2 * ff + 1).astype(jnp.bfloat16)

        x16 = jnp.concatenate(
            [x[...].astype(jnp.bfloat16) for x in (x0, x1, x2, x3)],
            axis=1)
        t = lax.dot_general(x16, i_scr[...], (((0,), (0,)), ((), ())),
                            preferred_element_type=jnp.float32)
        t16 = t.astype(jnp.bfloat16)
        lo = lax.dot_general(t16, fe[...], (((1,), (0,)), ((), ())),
                             preferred_element_type=jnp.float32)
        hi = lax.dot_general(t16, fo[...], (((1,), (0,)), ((), ())),
                             preferred_element_type=jnp.float32)
        lo_w = lax.bitcast_convert_type(
            lo.astype(jnp.bfloat16), jnp.int16).astype(jnp.int32) & 0xFFFF
        hi_w = lax.shift_left(
            lax.bitcast_convert_type(
                hi.astype(jnp.bfloat16), jnp.int16).astype(jnp.int32), 16)
        w = lo_w | hi_w
        out_ref[...] = jnp.concatenate(
            [w[q * _QW:(q + 1) * _QW, :] for q in range(4)], axis=1)

    return pl.pallas_call(
        body,
        grid=(_NBLK,),
        in_specs=[
            pl.BlockSpec(
                (_D, _QW),
                lambda i, q=q: (0, jnp.minimum(q * _NBLK + i,
                                               (_VOCAB - 1) // _QW)))
            for q in range(4)
        ],
        out_specs=pl.BlockSpec((_QW, 128), lambda i: (i, 0)),
        out_shape=jax.ShapeDtypeStruct((_VPAD // 4, 128), jnp.int32),
        scratch_shapes=[
            pltpu.VMEM((_D, _D), jnp.bfloat16),
            pltpu.VMEM((_D, 32), jnp.bfloat16),
            pltpu.VMEM((_D, 32), jnp.bfloat16),
        ],
    )(embedT, embedT, embedT, embedT)


def _sc_pool(idx2d, table_w):
    mesh = plsc.VectorSubcoreMesh(core_axis_name="c", subcore_axis_name="s")

    @functools.partial(
        pl.kernel,
        mesh=mesh,
        compiler_params=pltpu.CompilerParams(
            use_tc_tiling_on_sc=False, needs_layout_passes=False),
        out_type=jax.ShapeDtypeStruct((_B, _D), jnp.float32),
        scratch_types=[
            pltpu.VMEM((_BPW, _CTX), jnp.int32),
            pltpu.VMEM((2, _CB, _CTX, _D // 2), jnp.int32),
            pltpu.VMEM((2, _CB, _D), jnp.float32),
            pltpu.SemaphoreType.DMA,
            pltpu.SemaphoreType.DMA,
            pltpu.SemaphoreType.DMA,
            pltpu.SemaphoreType.DMA,
        ],
    )
    def k(idx_hbm, table_hbm, out_hbm, idx_v, rows_v, acc_v, g0, g1, o0, o1):
        wid = lax.axis_index("s") * _NC + lax.axis_index("c")
        b0w = wid * _BPW
        gsem = (g0, g1)
        osem = (o0, o1)

        pltpu.sync_copy(idx_hbm.at[pl.ds(b0w, _BPW)], idx_v)

        def gather_descs(i, par):
            return [
                pltpu.make_async_copy(
                    table_hbm.at[idx_v.at[i * _CB + bb]],
                    rows_v.at[par, bb],
                    gsem[par],
                )
                for bb in range(_CB)
            ]

        def fire(i, par):
            for d in gather_descs(i, par):
                d.start()

        def drain(i, par):
            for d in gather_descs(i, par):
                d.wait()

        def pool(i, par):
            for bb in range(_CB):
                zeros = jnp.zeros((16,), jnp.float32)

                def ctx_body(c, acc, bb=bb, par=par):
                    r = c * 2
                    a0, a1, a2, a3 = acc
                    for u in range(2):
                        lohalf = plsc.bitcast(
                            rows_v[par, bb, r + u, pl.ds(0, 16)], jnp.bfloat16)
                        hihalf = plsc.bitcast(
                            rows_v[par, bb, r + u, pl.ds(16, 16)], jnp.bfloat16)
                        e0, o0_ = plsc.unpack(
                            lohalf, format=plsc.PackFormat.INTERLEAVED)
                        e1, o1_ = plsc.unpack(
                            hihalf, format=plsc.PackFormat.INTERLEAVED)
                        a0 = a0 + e0
                        a1 = a1 + o0_
                        a2 = a2 + e1
                        a3 = a3 + o1_
                    return (a0, a1, a2, a3)

                a0, a1, a2, a3 = lax.fori_loop(
                    0, _CTX // 2, ctx_body, (zeros, zeros, zeros, zeros))
                acc_v[par, bb, pl.ds(0, 16)] = a0
                acc_v[par, bb, pl.ds(16, 16)] = a1
                acc_v[par, bb, pl.ds(32, 16)] = a2
                acc_v[par, bb, pl.ds(48, 16)] = a3

        def out_desc(i, par):
            return pltpu.make_async_copy(
                acc_v.at[par],
                out_hbm.at[pl.ds(b0w + i * _CB, _CB)],
                osem[par],
            )

        fire(0, 0)

        def pair_body(p, carry):
            for q in range(2):
                i = 2 * p + q
                par = q
                drain(i, par)

                @pl.when(i + 1 < _NCHUNK)
                def _():
                    fire(i + 1, 1 - par)

                @pl.when(i >= 2)
                def _():
                    out_desc(i - 2, par).wait()

                pool(i, par)
                out_desc(i, par).start()
            return carry

        lax.fori_loop(0, _NCHUNK // 2, pair_body, 0)

        out_desc(_NCHUNK - 2, 0).wait()
        out_desc(_NCHUNK - 1, 1).wait()

    return k(idx2d, table_w)


def _tc_linear(pooled, Wp, b2d):
    BB = 2048

    def body(x_ref, w_ref, b_ref, o_ref):
        o_ref[...] = lax.dot_general(
            x_ref[...], w_ref[...], (((1,), (1,)), ((), ())),
            preferred_element_type=jnp.float32,
        ) + b_ref[...]

    return pl.pallas_call(
        body,
        grid=(_B // BB,),
        in_specs=[
            pl.BlockSpec((BB, _D), lambda i: (i, 0)),
            pl.BlockSpec((_ODIM, _D), lambda i: (0, 0)),
            pl.BlockSpec((1, _ODIM), lambda i: (0, 0)),
        ],
        out_specs=pl.BlockSpec((BB, _ODIM), lambda i: (i, 0)),
        out_shape=jax.ShapeDtypeStruct((_B, _ODIM), jnp.float32),
    )(pooled, Wp, b2d)


def kernel(inputs, embed, W, b):
    packed = _tc_relayout(jnp.transpose(embed))
    idx = inputs.astype(jnp.int32)
    idxp = 4 * (idx % _VQ) + idx // _VQ
    pooled = _sc_pool(idxp, packed.reshape(_VPAD, _D // 2))
    Wp = W[:, _PERM]
    return _tc_linear(pooled, Wp, b.reshape(1, _ODIM))


# speedup vs baseline: 4.1484x; 1.1672x over previous
import functools

import jax
import jax.numpy as jnp
import numpy as _np
from jax import lax
from jax.experimental import pallas as pl
from jax.experimental.pallas import tpu as pltpu
from jax.experimental.pallas import tpu_sc as plsc

_VOCAB = 1000000
_D = 64
_ODIM = 128
_B = 16384
_CTX = 50

_NC = 2
_NS = 16
_NW = _NC * _NS
_BPW = _B // _NW
_CB = 8
_NCHUNK = _BPW // _CB

_QW = 2048
_NBLK = 123
_VQ = _NBLK * _QW
_VPAD = 4 * _VQ

_PERM = _np.concatenate([
    _np.arange(0, 32, 2), _np.arange(1, 32, 2),
    _np.arange(32, 64, 2), _np.arange(33, 64, 2),
])


def _tc_relayout(embedT):
    def body(x0, x1, x2, x3, out_ref, i_scr, fe, fo):
        i = pl.program_id(0)

        @pl.when(i == 0)
        def _():
            a = lax.broadcasted_iota(jnp.int32, (_D, _D), 0)
            b = lax.broadcasted_iota(jnp.int32, (_D, _D), 1)
            i_scr[...] = (a == b).astype(jnp.bfloat16)
            jf = lax.broadcasted_iota(jnp.int32, (_D, 32), 0)
            ff = lax.broadcasted_iota(jnp.int32, (_D, 32), 1)
            fe[...] = (jf == 2 * ff).astype(jnp.bfloat16)
            fo[...] = (jf == 2 * ff + 1).astype(jnp.bfloat16)

        x16 = jnp.concatenate(
            [x[...].astype(jnp.bfloat16) for x in (x0, x1, x2, x3)],
            axis=1)
        t = lax.dot_general(x16, i_scr[...], (((0,), (0,)), ((), ())),
                            preferred_element_type=jnp.float32)
        t16 = t.astype(jnp.bfloat16)
        lo = lax.dot_general(t16, fe[...], (((1,), (0,)), ((), ())),
                             preferred_element_type=jnp.float32)
        hi = lax.dot_general(t16, fo[...], (((1,), (0,)), ((), ())),
                             preferred_element_type=jnp.float32)
        lo_w = lax.bitcast_convert_type(
            lo.astype(jnp.bfloat16), jnp.int16).astype(jnp.int32) & 0xFFFF
        hi_w = lax.shift_left(
            lax.bitcast_convert_type(
                hi.astype(jnp.bfloat16), jnp.int16).astype(jnp.int32), 16)
        w = lo_w | hi_w
        out_ref[...] = jnp.concatenate(
            [w[q * _QW:(q + 1) * _QW, :] for q in range(4)], axis=1)

    return pl.pallas_call(
        body,
        grid=(_NBLK,),
        in_specs=[
            pl.BlockSpec(
                (_D, _QW),
                lambda i, q=q: (0, jnp.minimum(q * _NBLK + i,
                                               (_VOCAB - 1) // _QW)))
            for q in range(4)
        ],
        out_specs=pl.BlockSpec((_QW, 128), lambda i: (i, 0)),
        out_shape=jax.ShapeDtypeStruct((_VPAD // 4, 128), jnp.int32),
        scratch_shapes=[
            pltpu.VMEM((_D, _D), jnp.bfloat16),
            pltpu.VMEM((_D, 32), jnp.bfloat16),
            pltpu.VMEM((_D, 32), jnp.bfloat16),
        ],
    )(embedT, embedT, embedT, embedT)


def _sc_pool(idx2d, table_w):
    mesh = plsc.VectorSubcoreMesh(core_axis_name="c", subcore_axis_name="s")

    @functools.partial(
        pl.kernel,
        mesh=mesh,
        compiler_params=pltpu.CompilerParams(
            use_tc_tiling_on_sc=False, needs_layout_passes=False),
        out_type=jax.ShapeDtypeStruct((_B, _D), jnp.float32),
        scratch_types=[
            pltpu.VMEM((_BPW, _CTX), jnp.int32),
            pltpu.VMEM((2, _CB, _CTX, _D // 2), jnp.int32),
            pltpu.VMEM((2, _CB, _D), jnp.float32),
            pltpu.SemaphoreType.DMA,
            pltpu.SemaphoreType.DMA,
            pltpu.SemaphoreType.DMA,
            pltpu.SemaphoreType.DMA,
        ],
    )
    def k(idx_hbm, table_hbm, out_hbm, idx_v, rows_v, acc_v, g0, g1, o0, o1):
        wid = lax.axis_index("s") * _NC + lax.axis_index("c")
        b0w = wid * _BPW
        gsem = (g0, g1)
        osem = (o0, o1)

        pltpu.sync_copy(idx_hbm.at[pl.ds(b0w, _BPW)], idx_v)

        def gather_descs(i, par):
            return [
                pltpu.make_async_copy(
                    table_hbm.at[idx_v.at[i * _CB + bb]],
                    rows_v.at[par, bb],
                    gsem[par],
                )
                for bb in range(_CB)
            ]

        def fire(i, par):
            for d in gather_descs(i, par):
                d.start()

        def drain(i, par):
            for d in gather_descs(i, par):
                d.wait()

        def pool(i, par):
            for bb in range(_CB):
                zeros = jnp.zeros((16,), jnp.float32)

                def ctx_body(c, acc, bb=bb, par=par):
                    r = c * 2
                    a0, a1, a2, a3 = acc
                    for u in range(2):
                        lohalf = plsc.bitcast(
                            rows_v[par, bb, r + u, pl.ds(0, 16)], jnp.bfloat16)
                        hihalf = plsc.bitcast(
                            rows_v[par, bb, r + u, pl.ds(16, 16)], jnp.bfloat16)
                        e0, o0_ = plsc.unpack(
                            lohalf, format=plsc.PackFormat.INTERLEAVED)
                        e1, o1_ = plsc.unpack(
                            hihalf, format=plsc.PackFormat.INTERLEAVED)
                        a0 = a0 + e0
                        a1 = a1 + o0_
                        a2 = a2 + e1
                        a3 = a3 + o1_
                    return (a0, a1, a2, a3)

                a0, a1, a2, a3 = lax.fori_loop(
                    0, _CTX // 2, ctx_body, (zeros, zeros, zeros, zeros))
                acc_v[par, bb, pl.ds(0, 16)] = a0
                acc_v[par, bb, pl.ds(16, 16)] = a1
                acc_v[par, bb, pl.ds(32, 16)] = a2
                acc_v[par, bb, pl.ds(48, 16)] = a3

        def out_desc(i, par):
            return pltpu.make_async_copy(
                acc_v.at[par],
                out_hbm.at[pl.ds(b0w + i * _CB, _CB)],
                osem[par],
            )

        fire(0, 0)

        def pair_body(p, carry):
            for q in range(2):
                i = 2 * p + q
                par = q
                drain(i, par)

                @pl.when(i + 1 < _NCHUNK)
                def _():
                    fire(i + 1, 1 - par)

                @pl.when(i >= 2)
                def _():
                    out_desc(i - 2, par).wait()

                pool(i, par)
                out_desc(i, par).start()
            return carry

        lax.fori_loop(0, _NCHUNK // 2, pair_body, 0)

        out_desc(_NCHUNK - 2, 0).wait()
        out_desc(_NCHUNK - 1, 1).wait()

    return k(idx2d, table_w)


def _tc_linear(pooled, Wp, b2d):
    BB = 2048

    def body(x_ref, w_ref, b_ref, o_ref):
        o_ref[...] = lax.dot_general(
            x_ref[...], w_ref[...], (((1,), (1,)), ((), ())),
            preferred_element_type=jnp.float32,
        ) + b_ref[...]

    return pl.pallas_call(
        body,
        grid=(_B // BB,),
        in_specs=[
            pl.BlockSpec((BB, _D), lambda i: (i, 0)),
            pl.BlockSpec((_ODIM, _D), lambda i: (0, 0)),
            pl.BlockSpec((1, _ODIM), lambda i: (0, 0)),
        ],
        out_specs=pl.BlockSpec((BB, _ODIM), lambda i: (i, 0)),
        out_shape=jax.ShapeDtypeStruct((_B, _ODIM), jnp.float32),
    )(pooled, Wp, b2d)


def kernel(inputs, embed, W, b):
    packed = _tc_relayout(jnp.transpose(embed))
    idx = inputs.astype(jnp.int32)
    idxp = 4 * (idx % _VQ) + idx // _VQ
    pooled = _sc_pool(idxp, packed.reshape(_VPAD, _D // 2))
    Wp = W[:, _PERM]
    return _tc_linear(pooled, Wp, b.reshape(1, _ODIM))


# speedup vs baseline: 4.4135x; 1.0639x over previous
import functools

import jax
import jax.numpy as jnp
import numpy as _np
from jax import lax
from jax.experimental import pallas as pl
from jax.experimental.pallas import tpu as pltpu
from jax.experimental.pallas import tpu_sc as plsc

_VOCAB = 1000000
_D = 64
_ODIM = 128
_B = 16384
_CTX = 50

_NC = 2
_NS = 16
_NW = _NC * _NS
_BPW = _B // _NW
_CB = 8
_NCHUNK = _BPW // _CB

_QW = 4096
_NBLK = 62
_VQ = _NBLK * _QW
_VPAD = 4 * _VQ

_PERM = _np.concatenate([
    _np.arange(0, 32, 2), _np.arange(1, 32, 2),
    _np.arange(32, 64, 2), _np.arange(33, 64, 2),
])


def _tc_relayout(embedT):
    def body(x0, x1, x2, x3, out_ref, i_scr, fe, fo):
        i = pl.program_id(0)

        @pl.when(i == 0)
        def _():
            a = lax.broadcasted_iota(jnp.int32, (_D, _D), 0)
            b = lax.broadcasted_iota(jnp.int32, (_D, _D), 1)
            i_scr[...] = (a == b).astype(jnp.bfloat16)
            jf = lax.broadcasted_iota(jnp.int32, (_D, 32), 0)
            ff = lax.broadcasted_iota(jnp.int32, (_D, 32), 1)
            fe[...] = (jf == 2 * ff).astype(jnp.bfloat16)
            fo[...] = (jf == 2 * ff + 1).astype(jnp.bfloat16)

        x16 = jnp.concatenate(
            [x[...].astype(jnp.bfloat16) for x in (x0, x1, x2, x3)],
            axis=1)
        t = lax.dot_general(x16, i_scr[...], (((0,), (0,)), ((), ())),
                            preferred_element_type=jnp.float32)
        t16 = t.astype(jnp.bfloat16)
        lo = lax.dot_general(t16, fe[...], (((1,), (0,)), ((), ())),
                             preferred_element_type=jnp.float32)
        hi = lax.dot_general(t16, fo[...], (((1,), (0,)), ((), ())),
                             preferred_element_type=jnp.float32)
        lo_w = lax.bitcast_convert_type(
            lo.astype(jnp.bfloat16), jnp.int16).astype(jnp.int32) & 0xFFFF
        hi_w = lax.shift_left(
            lax.bitcast_convert_type(
                hi.astype(jnp.bfloat16), jnp.int16).astype(jnp.int32), 16)
        w = lo_w | hi_w
        out_ref[...] = jnp.concatenate(
            [w[q * _QW:(q + 1) * _QW, :] for q in range(4)], axis=1)

    return pl.pallas_call(
        body,
        grid=(_NBLK,),
        in_specs=[
            pl.BlockSpec(
                (_D, _QW),
                lambda i, q=q: (0, jnp.minimum(q * _NBLK + i,
                                               (_VOCAB - 1) // _QW)))
            for q in range(4)
        ],
        out_specs=pl.BlockSpec((_QW, 128), lambda i: (i, 0)),
        out_shape=jax.ShapeDtypeStruct((_VPAD // 4, 128), jnp.int32),
        scratch_shapes=[
            pltpu.VMEM((_D, _D), jnp.bfloat16),
            pltpu.VMEM((_D, 32), jnp.bfloat16),
            pltpu.VMEM((_D, 32), jnp.bfloat16),
        ],
    )(embedT, embedT, embedT, embedT)


def _sc_pool(idx2d, table_w):
    mesh = plsc.VectorSubcoreMesh(core_axis_name="c", subcore_axis_name="s")

    @functools.partial(
        pl.kernel,
        mesh=mesh,
        compiler_params=pltpu.CompilerParams(
            use_tc_tiling_on_sc=False, needs_layout_passes=False),
        out_type=jax.ShapeDtypeStruct((_B, _D), jnp.float32),
        scratch_types=[
            pltpu.VMEM((_BPW, _CTX), jnp.int32),
            pltpu.VMEM((2, _CB, _CTX, _D // 2), jnp.int32),
            pltpu.VMEM((2, _CB, _D), jnp.float32),
            pltpu.SemaphoreType.DMA,
            pltpu.SemaphoreType.DMA,
            pltpu.SemaphoreType.DMA,
            pltpu.SemaphoreType.DMA,
        ],
    )
    def k(idx_hbm, table_hbm, out_hbm, idx_v, rows_v, acc_v, g0, g1, o0, o1):
        wid = lax.axis_index("s") * _NC + lax.axis_index("c")
        b0w = wid * _BPW
        gsem = (g0, g1)
        osem = (o0, o1)

        pltpu.sync_copy(idx_hbm.at[pl.ds(b0w, _BPW)], idx_v)

        def gather_descs(i, par):
            return [
                pltpu.make_async_copy(
                    table_hbm.at[idx_v.at[i * _CB + bb]],
                    rows_v.at[par, bb],
                    gsem[par],
                )
                for bb in range(_CB)
            ]

        def fire(i, par):
            for d in gather_descs(i, par):
                d.start()

        def drain(i, par):
            for d in gather_descs(i, par):
                d.wait()

        def pool(i, par):
            for bb in range(_CB):
                zeros = jnp.zeros((16,), jnp.float32)

                def ctx_body(c, acc, bb=bb, par=par):
                    r = c * 2
                    a0, a1, a2, a3 = acc
                    for u in range(2):
                        lohalf = plsc.bitcast(
                            rows_v[par, bb, r + u, pl.ds(0, 16)], jnp.bfloat16)
                        hihalf = plsc.bitcast(
                            rows_v[par, bb, r + u, pl.ds(16, 16)], jnp.bfloat16)
                        e0, o0_ = plsc.unpack(
                            lohalf, format=plsc.PackFormat.INTERLEAVED)
                        e1, o1_ = plsc.unpack(
                            hihalf, format=plsc.PackFormat.INTERLEAVED)
                        a0 = a0 + e0
                        a1 = a1 + o0_
                        a2 = a2 + e1
                        a3 = a3 + o1_
                    return (a0, a1, a2, a3)

                a0, a1, a2, a3 = lax.fori_loop(
                    0, _CTX // 2, ctx_body, (zeros, zeros, zeros, zeros))
                acc_v[par, bb, pl.ds(0, 16)] = a0
                acc_v[par, bb, pl.ds(16, 16)] = a1
                acc_v[par, bb, pl.ds(32, 16)] = a2
                acc_v[par, bb, pl.ds(48, 16)] = a3

        def out_desc(i, par):
            return pltpu.make_async_copy(
                acc_v.at[par],
                out_hbm.at[pl.ds(b0w + i * _CB, _CB)],
                osem[par],
            )

        fire(0, 0)

        def pair_body(p, carry):
            for q in range(2):
                i = 2 * p + q
                par = q
                drain(i, par)

                @pl.when(i + 1 < _NCHUNK)
                def _():
                    fire(i + 1, 1 - par)

                @pl.when(i >= 2)
                def _():
                    out_desc(i - 2, par).wait()

                pool(i, par)
                out_desc(i, par).start()
            return carry

        lax.fori_loop(0, _NCHUNK // 2, pair_body, 0)

        out_desc(_NCHUNK - 2, 0).wait()
        out_desc(_NCHUNK - 1, 1).wait()

    return k(idx2d, table_w)


def _tc_linear(pooled, Wp, b2d):
    BB = 2048

    def body(x_ref, w_ref, b_ref, o_ref):
        o_ref[...] = lax.dot_general(
            x_ref[...], w_ref[...], (((1,), (1,)), ((), ())),
            preferred_element_type=jnp.float32,
        ) + b_ref[...]

    return pl.pallas_call(
        body,
        grid=(_B // BB,),
        in_specs=[
            pl.BlockSpec((BB, _D), lambda i: (i, 0)),
            pl.BlockSpec((_ODIM, _D), lambda i: (0, 0)),
            pl.BlockSpec((1, _ODIM), lambda i: (0, 0)),
        ],
        out_specs=pl.BlockSpec((BB, _ODIM), lambda i: (i, 0)),
        out_shape=jax.ShapeDtypeStruct((_B, _ODIM), jnp.float32),
    )(pooled, Wp, b2d)


def kernel(inputs, embed, W, b):
    packed = _tc_relayout(jnp.transpose(embed))
    idx = inputs.astype(jnp.int32)
    idxp = 4 * (idx % _VQ) + idx // _VQ
    pooled = _sc_pool(idxp, packed.reshape(_VPAD, _D // 2))
    Wp = W[:, _PERM]
    return _tc_linear(pooled, Wp, b.reshape(1, _ODIM))
